# baseline (speedup 1.0000x reference)
import jax
import jax.numpy as jnp
from jax import lax
from jax.experimental import pallas as pl
from jax.experimental.pallas import tpu as pltpu
from jax.experimental.pallas import tpu_sc as plsc

N_ENV = 128
N_SRV = 2048
LANES = 16
N_CHUNKS = N_ENV // LANES
BLK_COLS = 128


def _sc_route_body(idx_hbm, cpu_hbm, ram_hbm, acpu_hbm, ccpu_hbm, aram_hbm,
                   cram_hbm, heu_hbm, idxv, cpuv, ramv, gb0, gb1,
                   a0v, c0v, r0v, d0v, a1v, c1v, r1v, d1v, heu_v, sem_g):
    sid = lax.axis_index("s")

    @pl.when(sid < N_CHUNKS)
    def _():
        base = sid * LANES
        pltpu.sync_copy(idx_hbm, idxv)

        lane = jnp.arange(LANES, dtype=jnp.int32)
        env2 = 2 * base + 2 * lane
        gb0[...] = plsc.load_gather(idxv, [env2])
        gb1[...] = plsc.load_gather(idxv, [env2 + 1])

        d = [pltpu.async_copy(acpu_hbm.at[gb0], a0v, sem_g),
             pltpu.async_copy(ccpu_hbm.at[gb0], c0v, sem_g),
             pltpu.async_copy(aram_hbm.at[gb0], r0v, sem_g),
             pltpu.async_copy(cram_hbm.at[gb0], d0v, sem_g),
             pltpu.async_copy(acpu_hbm.at[gb1], a1v, sem_g),
             pltpu.async_copy(ccpu_hbm.at[gb1], c1v, sem_g),
             pltpu.async_copy(aram_hbm.at[gb1], r1v, sem_g),
             pltpu.async_copy(cram_hbm.at[gb1], d1v, sem_g),
             pltpu.async_copy(cpu_hbm.at[pl.ds(base, LANES)], cpuv, sem_g),
             pltpu.async_copy(ram_hbm.at[pl.ds(base, LANES)], ramv, sem_g)]
        for cp in d:
            cp.wait()

        creq = cpuv[...]
        rreq = ramv[...]
        lb0 = (a0v[...] - creq) / c0v[...] + (r0v[...] - rreq) / d0v[...]
        lb1 = (a1v[...] - creq) / c1v[...] + (r1v[...] - rreq) / d1v[...]
        win1 = lb1 > lb0

        head = idxv[pl.ds(0, LANES)]
        neg = jnp.full((LANES,), -1, jnp.int32)
        cand0 = jnp.full((LANES,), jnp.max(jnp.where(lane == 0, head, neg)))
        cand1 = jnp.full((LANES,), jnp.max(jnp.where(lane == 2, head, neg)))
        heu_v[...] = jnp.where(win1, cand1, cand0)
        pltpu.sync_copy(heu_v, heu_hbm.at[pl.ds(base, LANES)])


def _sc_route(idx_flat, cpu_req, ram_req, acpu, ccpu, aram, cram):
    mesh = plsc.VectorSubcoreMesh(core_axis_name="c", subcore_axis_name="s",
                                  num_cores=1)
    return pl.kernel(
        _sc_route_body,
        out_type=jax.ShapeDtypeStruct((N_ENV,), jnp.int32),
        mesh=mesh,
        compiler_params=pltpu.CompilerParams(needs_layout_passes=False),
        scratch_types=[
            pltpu.VMEM((N_ENV * 2,), jnp.int32),
            pltpu.VMEM((LANES,), jnp.float32),
            pltpu.VMEM((LANES,), jnp.float32),
            pltpu.VMEM((LANES,), jnp.int32),
            pltpu.VMEM((LANES,), jnp.int32),
            pltpu.VMEM((LANES,), jnp.float32),
            pltpu.VMEM((LANES,), jnp.float32),
            pltpu.VMEM((LANES,), jnp.float32),
            pltpu.VMEM((LANES,), jnp.float32),
            pltpu.VMEM((LANES,), jnp.float32),
            pltpu.VMEM((LANES,), jnp.float32),
            pltpu.VMEM((LANES,), jnp.float32),
            pltpu.VMEM((LANES,), jnp.float32),
            pltpu.VMEM((LANES,), jnp.int32),
            pltpu.SemaphoreType.DMA,
        ],
    )(idx_flat, cpu_req, ram_req, acpu, ccpu, aram, cram)


ROWS_PER_BLK = 16
GRID_A = N_ENV // ROWS_PER_BLK


def _tc_copymax_body(x_ref, o_ref, mx_ref):
    x = x_ref[...]
    o_ref[...] = x
    mx_ref[...] = jnp.max(x, axis=1, keepdims=True)


def _tc_copymax(x):
    return pl.pallas_call(
        _tc_copymax_body,
        grid=(GRID_A,),
        in_specs=[pl.BlockSpec((ROWS_PER_BLK, N_SRV), lambda i: (i, 0))],
        out_specs=[pl.BlockSpec((ROWS_PER_BLK, N_SRV), lambda i: (i, 0)),
                   pl.BlockSpec((ROWS_PER_BLK, 1), lambda i: (i, 0))],
        out_shape=[jax.ShapeDtypeStruct((N_ENV, N_SRV), jnp.float32),
                   jax.ShapeDtypeStruct((N_ENV, 1), jnp.float32)],
    )(x)


def _tc_fixup_body(sblk_ref, o_in_ref, heu_ref, mx_ref, o_ref):
    j = sblk_ref[pl.program_id(0)]
    cols = j * BLK_COLS + lax.broadcasted_iota(jnp.int32, (N_ENV, BLK_COLS), 1)
    o_ref[...] = jnp.where(cols == heu_ref[...], mx_ref[...], o_in_ref[...])


def _tc_fixup(sblk, out0, heu2d, mx):
    return pl.pallas_call(
        _tc_fixup_body,
        grid_spec=pltpu.PrefetchScalarGridSpec(
            num_scalar_prefetch=1,
            grid=(2,),
            in_specs=[
                pl.BlockSpec((N_ENV, BLK_COLS), lambda i, sblk: (0, sblk[i])),
                pl.BlockSpec((N_ENV, 1), lambda i, sblk: (0, 0)),
                pl.BlockSpec((N_ENV, 1), lambda i, sblk: (0, 0)),
            ],
            out_specs=pl.BlockSpec((N_ENV, BLK_COLS),
                                   lambda i, sblk: (0, sblk[i])),
        ),
        out_shape=jax.ShapeDtypeStruct((N_ENV, N_SRV), jnp.float32),
        input_output_aliases={1: 0},
    )(sblk, out0, heu2d, mx)


@jax.jit
def _run(x, idx_flat, cpu_req, ram_req, acpu, ccpu, aram, cram):
    heu = _sc_route(idx_flat, cpu_req, ram_req, acpu, ccpu, aram, cram)
    out0, mx = _tc_copymax(x)
    sblk = jnp.stack([idx_flat[0] // BLK_COLS, idx_flat[2] // BLK_COLS])
    return _tc_fixup(sblk, out0, heu.reshape(N_ENV, 1), mx)


def kernel(x, cur_vnf_cpu_req, cur_vnf_ram_req, availCPU, CPUcap, availRAM,
           RAMcap, sampled_indexes):
    idx = sampled_indexes.astype(jnp.int32).reshape(-1)
    return _run(x, idx, cur_vnf_cpu_req, cur_vnf_ram_req,
                availCPU, CPUcap, availRAM, RAMcap)


# speedup vs baseline: 1.2293x; 1.2293x over previous
import jax
import jax.numpy as jnp
from jax import lax
from jax.experimental import pallas as pl
from jax.experimental.pallas import tpu as pltpu
from jax.experimental.pallas import tpu_sc as plsc

N_ENV = 128
N_SRV = 2048
LANES = 16
N_WORKERS = 16
ROWS_PER_W = N_ENV // N_WORKERS
FLAT_PER_W = ROWS_PER_W * N_SRV


def _sc_body(x_hbm, idx_hbm, cpu_hbm, ram_hbm, acpu_hbm, ccpu_hbm, aram_hbm,
             cram_hbm, out_hbm, xv, idxv, cpuv, ramv, gb0, gb1,
             a0v, c0v, r0v, d0v, a1v, c1v, r1v, d1v, sem_x, sem_g):
    wid = lax.axis_index("s")
    fbase = wid * FLAT_PER_W

    cp_x = pltpu.async_copy(x_hbm.at[pl.ds(fbase, FLAT_PER_W)], xv, sem_x)

    pltpu.sync_copy(idx_hbm, idxv)

    lane = jnp.arange(LANES, dtype=jnp.int32)
    row = jnp.minimum(lane, ROWS_PER_W - 1)
    env2 = 2 * (ROWS_PER_W * wid) + 2 * row
    i0 = plsc.load_gather(idxv, [env2])
    i1 = plsc.load_gather(idxv, [env2 + 1])
    gb0[...] = i0
    gb1[...] = i1

    d = [pltpu.async_copy(acpu_hbm.at[gb0], a0v, sem_g),
         pltpu.async_copy(ccpu_hbm.at[gb0], c0v, sem_g),
         pltpu.async_copy(aram_hbm.at[gb0], r0v, sem_g),
         pltpu.async_copy(cram_hbm.at[gb0], d0v, sem_g),
         pltpu.async_copy(acpu_hbm.at[gb1], a1v, sem_g),
         pltpu.async_copy(ccpu_hbm.at[gb1], c1v, sem_g),
         pltpu.async_copy(aram_hbm.at[gb1], r1v, sem_g),
         pltpu.async_copy(cram_hbm.at[gb1], d1v, sem_g),
         pltpu.async_copy(cpu_hbm, cpuv, sem_g),
         pltpu.async_copy(ram_hbm, ramv, sem_g)]
    for cp in d:
        cp.wait()

    creq = plsc.load_gather(cpuv, [ROWS_PER_W * wid + row])
    rreq = plsc.load_gather(ramv, [ROWS_PER_W * wid + row])
    lb0 = (a0v[...] - creq) / c0v[...] + (r0v[...] - rreq) / d0v[...]
    lb1 = (a1v[...] - creq) / c1v[...] + (r1v[...] - rreq) / d1v[...]
    win1 = lb1 > lb0

    head = idxv[pl.ds(0, LANES)]
    neg = jnp.full((LANES,), -1, jnp.int32)
    cand0 = jnp.full((LANES,), jnp.max(jnp.where(lane == 0, head, neg)))
    cand1 = jnp.full((LANES,), jnp.max(jnp.where(lane == 2, head, neg)))
    heu = jnp.where(win1, cand1, cand0)

    cp_x.wait()

    ninf = jnp.full((LANES,), -jnp.inf, jnp.float32)

    def _max_body(off, ms):
        return tuple(
            jnp.maximum(m, xv[pl.ds(r * N_SRV + off, LANES)])
            for r, m in enumerate(ms))

    maxes = plsc.parallel_loop(
        0, N_SRV, LANES, unroll=4,
        carry=(ninf,) * ROWS_PER_W)(_max_body)

    mx = jnp.full((LANES,), jnp.max(maxes[0]))
    for r in range(1, ROWS_PER_W):
        mx = jnp.where(lane == r, jnp.max(maxes[r]), mx)

    pos = row * N_SRV + heu
    plsc.store_scatter(xv, [pos], mx, mask=lane < ROWS_PER_W)

    pltpu.sync_copy(xv, out_hbm.at[pl.ds(fbase, FLAT_PER_W)])


@jax.jit
def _run(x_flat, idx_flat, cpu_req, ram_req, acpu, ccpu, aram, cram):
    mesh = plsc.VectorSubcoreMesh(core_axis_name="c", subcore_axis_name="s",
                                  num_cores=1)
    return pl.kernel(
        _sc_body,
        out_type=jax.ShapeDtypeStruct((N_ENV * N_SRV,), jnp.float32),
        mesh=mesh,
        compiler_params=pltpu.CompilerParams(needs_layout_passes=False,
                                             skip_device_barrier=True),
        scratch_types=[
            pltpu.VMEM((FLAT_PER_W,), jnp.float32),
            pltpu.VMEM((N_ENV * 2,), jnp.int32),
            pltpu.VMEM((N_ENV,), jnp.float32),
            pltpu.VMEM((N_ENV,), jnp.float32),
            pltpu.VMEM((LANES,), jnp.int32),
            pltpu.VMEM((LANES,), jnp.int32),
            pltpu.VMEM((LANES,), jnp.float32),
            pltpu.VMEM((LANES,), jnp.float32),
            pltpu.VMEM((LANES,), jnp.float32),
            pltpu.VMEM((LANES,), jnp.float32),
            pltpu.VMEM((LANES,), jnp.float32),
            pltpu.VMEM((LANES,), jnp.float32),
            pltpu.VMEM((LANES,), jnp.float32),
            pltpu.VMEM((LANES,), jnp.float32),
            pltpu.SemaphoreType.DMA,
            pltpu.SemaphoreType.DMA,
        ],
    )(x_flat, idx_flat, cpu_req, ram_req, acpu, ccpu, aram, cram)


def kernel(x, cur_vnf_cpu_req, cur_vnf_ram_req, availCPU, CPUcap, availRAM,
           RAMcap, sampled_indexes):
    idx = sampled_indexes.astype(jnp.int32).reshape(-1)
    out = _run(x.reshape(-1), idx, cur_vnf_cpu_req, cur_vnf_ram_req,
               availCPU, CPUcap, availRAM, RAMcap)
    return out.reshape(N_ENV, N_SRV)


# speedup vs baseline: 1.4260x; 1.1600x over previous
import jax
import jax.numpy as jnp
from jax import lax
from jax.experimental import pallas as pl
from jax.experimental.pallas import tpu as pltpu
from jax.experimental.pallas import tpu_sc as plsc

N_ENV = 128
N_SRV = 2048
LANES = 16
N_WORKERS = 16
ROWS_PER_W = N_ENV // N_WORKERS
FLAT_PER_W = ROWS_PER_W * N_SRV


def _sc_body(x_hbm, idx_hbm, cpu_hbm, ram_hbm, acpu_hbm, ccpu_hbm, aram_hbm,
             cram_hbm, out_hbm, xv, idxv, cpuv, ramv, gb0, gb1,
             a0v, c0v, r0v, d0v, a1v, c1v, r1v, d1v, sem_x, sem_g):
    wid = lax.axis_index("s")
    rbase = wid * ROWS_PER_W

    cp_x = pltpu.async_copy(x_hbm.at[pl.ds(rbase, ROWS_PER_W)], xv, sem_x)

    pltpu.sync_copy(idx_hbm, idxv)

    lane = jnp.arange(LANES, dtype=jnp.int32)
    row = jnp.minimum(lane, ROWS_PER_W - 1)
    env2 = 2 * (ROWS_PER_W * wid) + 2 * row
    i0 = plsc.load_gather(idxv, [env2])
    i1 = plsc.load_gather(idxv, [env2 + 1])
    gb0[...] = i0
    gb1[...] = i1

    d = [pltpu.async_copy(acpu_hbm.at[gb0], a0v, sem_g),
         pltpu.async_copy(ccpu_hbm.at[gb0], c0v, sem_g),
         pltpu.async_copy(aram_hbm.at[gb0], r0v, sem_g),
         pltpu.async_copy(cram_hbm.at[gb0], d0v, sem_g),
         pltpu.async_copy(acpu_hbm.at[gb1], a1v, sem_g),
         pltpu.async_copy(ccpu_hbm.at[gb1], c1v, sem_g),
         pltpu.async_copy(aram_hbm.at[gb1], r1v, sem_g),
         pltpu.async_copy(cram_hbm.at[gb1], d1v, sem_g),
         pltpu.async_copy(cpu_hbm, cpuv, sem_g),
         pltpu.async_copy(ram_hbm, ramv, sem_g)]
    for cp in d:
        cp.wait()

    creq = plsc.load_gather(cpuv, [ROWS_PER_W * wid + row])
    rreq = plsc.load_gather(ramv, [ROWS_PER_W * wid + row])
    lb0 = (a0v[...] - creq) / c0v[...] + (r0v[...] - rreq) / d0v[...]
    lb1 = (a1v[...] - creq) / c1v[...] + (r1v[...] - rreq) / d1v[...]
    win1 = lb1 > lb0

    head = idxv[pl.ds(0, LANES)]
    neg = jnp.full((LANES,), -1, jnp.int32)
    cand0 = jnp.full((LANES,), jnp.max(jnp.where(lane == 0, head, neg)))
    cand1 = jnp.full((LANES,), jnp.max(jnp.where(lane == 2, head, neg)))
    heu = jnp.where(win1, cand1, cand0)

    cp_x.wait()

    ninf = jnp.full((LANES,), -jnp.inf, jnp.float32)

    def _max_body(off, ms):
        return tuple(
            jnp.maximum(m, xv[r, pl.ds(off, LANES)])
            for r, m in enumerate(ms))

    maxes = plsc.parallel_loop(
        0, N_SRV, LANES, unroll=4,
        carry=(ninf,) * ROWS_PER_W)(_max_body)

    mx = jnp.full((LANES,), jnp.max(maxes[0]))
    for r in range(1, ROWS_PER_W):
        mx = jnp.where(lane == r, jnp.max(maxes[r]), mx)

    plsc.store_scatter(xv, [row, heu], mx, mask=lane < ROWS_PER_W)

    pltpu.sync_copy(xv, out_hbm.at[pl.ds(rbase, ROWS_PER_W)])


@jax.jit
def _run(x, idx_flat, cpu_req, ram_req, acpu, ccpu, aram, cram):
    mesh = plsc.VectorSubcoreMesh(core_axis_name="c", subcore_axis_name="s",
                                  num_cores=1)
    return pl.kernel(
        _sc_body,
        out_type=jax.ShapeDtypeStruct((N_ENV, N_SRV), jnp.float32),
        mesh=mesh,
        compiler_params=pltpu.CompilerParams(needs_layout_passes=False,
                                             skip_device_barrier=True),
        scratch_types=[
            pltpu.VMEM((ROWS_PER_W, N_SRV), jnp.float32),
            pltpu.VMEM((N_ENV * 2,), jnp.int32),
            pltpu.VMEM((N_ENV,), jnp.float32),
            pltpu.VMEM((N_ENV,), jnp.float32),
            pltpu.VMEM((LANES,), jnp.int32),
            pltpu.VMEM((LANES,), jnp.int32),
            pltpu.VMEM((LANES,), jnp.float32),
            pltpu.VMEM((LANES,), jnp.float32),
            pltpu.VMEM((LANES,), jnp.float32),
            pltpu.VMEM((LANES,), jnp.float32),
            pltpu.VMEM((LANES,), jnp.float32),
            pltpu.VMEM((LANES,), jnp.float32),
            pltpu.VMEM((LANES,), jnp.float32),
            pltpu.VMEM((LANES,), jnp.float32),
            pltpu.SemaphoreType.DMA,
            pltpu.SemaphoreType.DMA,
        ],
    )(x, idx_flat, cpu_req, ram_req, acpu, ccpu, aram, cram)


def kernel(x, cur_vnf_cpu_req, cur_vnf_ram_req, availCPU, CPUcap, availRAM,
           RAMcap, sampled_indexes):
    idx = sampled_indexes.astype(jnp.int32).reshape(-1)
    return _run(x, idx, cur_vnf_cpu_req, cur_vnf_ram_req,
                availCPU, CPUcap, availRAM, RAMcap)
